# Initial kernel scaffold; baseline (speedup 1.0000x reference)
import jax
import jax.numpy as jnp
from jax.experimental import pallas as pl


def kernel(x, edge_index, edge_attr, W_ne, b_ne, W_ee, b_ee, eps, We, be, Wm1, bm1, Wm2, bm2, gamma, beta, Wo1, bo1, Wo2, bo2):
    raise NotImplementedError("write your pallas kernel here")


# speedup vs baseline: 1.3777x; 1.3777x over previous
import functools
import math

import jax
import jax.numpy as jnp
from jax import lax
from jax.experimental import pallas as pl
from jax.experimental.pallas import tpu as pltpu
from jax.experimental.pallas import tpu_sc as plsc

N = 10000
E = 320000
H = 128
NLAYERS = 3
NW = 16
EPW = E // NW
CHUNK = 40
NCHUNKS = EPW // CHUNK
GCH = 50
NGROUPS = NCHUNKS // GCH
RPT = N // 16
ZROWS = 25
NB = 1000
EB = 4000

_BN_SCALE = 1.0 / math.sqrt(1.0 + 1e-5)


def _node_enc_body(x_ref, w_ref, b_ref, o_ref):
    o_ref[...] = jnp.maximum(
        jnp.dot(x_ref[...], w_ref[...], preferred_element_type=jnp.float32)
        + b_ref[...], 0.0)


def _node_enc(x, w, b):
    return pl.pallas_call(
        _node_enc_body,
        grid=(N // NB,),
        in_specs=[
            pl.BlockSpec((NB, H), lambda i: (i, 0)),
            pl.BlockSpec((H, H), lambda i: (0, 0)),
            pl.BlockSpec((1, H), lambda i: (0, 0)),
        ],
        out_specs=pl.BlockSpec((NB, H), lambda i: (i, 0)),
        out_shape=jax.ShapeDtypeStruct((N, H), jnp.float32),
    )(x, w, b)


def _edge_e_body(ea_ref, wee_ref, we_ref, bee_ref, be_ref, e0_ref, e1_ref,
                 e2_ref):
    ea = ea_ref[...]
    outs = (e0_ref, e1_ref, e2_ref)
    for l in range(NLAYERS):
        wl = we_ref[l]
        wp = jnp.dot(wee_ref[...], wl, preferred_element_type=jnp.float32)
        bp = (jnp.dot(bee_ref[...], wl, preferred_element_type=jnp.float32)
              + be_ref[l][None, :])
        outs[l][...] = jnp.dot(ea, wp, preferred_element_type=jnp.float32) + bp


def _edge_e(edge_attr, w_ee, we, b_ee, be):
    d_edge = edge_attr.shape[1]
    espec = pl.BlockSpec((EB, H), lambda i: (i, 0))
    return pl.pallas_call(
        _edge_e_body,
        grid=(E // EB,),
        in_specs=[
            pl.BlockSpec((EB, d_edge), lambda i: (i, 0)),
            pl.BlockSpec((d_edge, H), lambda i: (0, 0)),
            pl.BlockSpec((NLAYERS, H, H), lambda i: (0, 0, 0)),
            pl.BlockSpec((1, H), lambda i: (0, 0)),
            pl.BlockSpec((NLAYERS, H), lambda i: (0, 0)),
        ],
        out_specs=[espec, espec, espec],
        out_shape=[jax.ShapeDtypeStruct((E, H), jnp.float32)] * NLAYERS,
    )(edge_attr, w_ee, we, b_ee, be)


@functools.cache
def _get_sc_msgpass():
    mesh = plsc.VectorSubcoreMesh(core_axis_name="c", subcore_axis_name="s",
                                  num_cores=1)
    return functools.partial(
        pl.kernel,
        out_type=jax.ShapeDtypeStruct((16, RPT, H), jnp.float32),
        mesh=mesh,
        scratch_types=[
            pltpu.VMEM((GCH, CHUNK), jnp.int32),
            pltpu.VMEM((GCH, CHUNK), jnp.int32),
            pltpu.VMEM((CHUNK, H), jnp.float32),
            pltpu.VMEM((ZROWS, H), jnp.float32),
            pltpu.VMEM_SHARED((N, H), jnp.float32),
            pltpu.SemaphoreType.DMA,
        ],
    )(_sc_msgpass_body)


def _sc_msgpass_body(h_hbm, src_hbm, dst_hbm, e_hbm, out_hbm, srcs_v, dsts_v,
                     buf, zbuf, aggr_sh, sem):
    sid = lax.axis_index("s")
    wid = sid

    def _zrow(r, carry):
        for j in range(H // 16):
            zbuf[r, pl.ds(j * 16, 16)] = jnp.zeros((16,), jnp.float32)
        return carry
    lax.fori_loop(0, ZROWS, _zrow, 0)
    for k in range(RPT // ZROWS):
        pltpu.sync_copy(zbuf, aggr_sh.at[pl.ds(sid * RPT + k * ZROWS, ZROWS), :])

    plsc.subcore_barrier()

    ebase = wid * EPW

    def _group(g, gcarry):
        pltpu.sync_copy(src_hbm.at[wid, g], srcs_v)
        pltpu.sync_copy(dst_hbm.at[wid, g], dsts_v)

        def _chunk(j, carry):
            base = ebase + (g * GCH + j) * CHUNK
            pltpu.sync_copy(e_hbm.at[pl.ds(base, CHUNK), :], buf)
            pltpu.async_copy(h_hbm.at[srcs_v.at[j]], buf, sem, add=True).wait()

            def _rrow(r, rc):
                for k in range(H // 16):
                    sl = pl.ds(k * 16, 16)
                    buf[r, sl] = jnp.maximum(buf[r, sl], 0.0)
                return rc
            lax.fori_loop(0, CHUNK, _rrow, 0)
            pltpu.sync_copy(buf, aggr_sh.at[dsts_v.at[j]], add=True)
            return carry

        lax.fori_loop(0, GCH, _chunk, 0)
        return gcarry

    lax.fori_loop(0, NGROUPS, _group, 0)
    plsc.subcore_barrier()

    pltpu.sync_copy(aggr_sh.at[pl.ds(sid * RPT, RPT), :], out_hbm.at[sid])


def _layer_body(h_ref, p_ref, eps_ref, wm1_ref, bm1_ref, wm2_ref, bm2_ref,
                g_ref, b_ref, o_ref):
    h = h_ref[...]
    out = (1.0 + eps_ref[0, 0]) * h + p_ref[...]
    t = jnp.maximum(
        jnp.dot(out, wm1_ref[...], preferred_element_type=jnp.float32)
        + bm1_ref[...], 0.0)
    out = (jnp.dot(t, wm2_ref[...], preferred_element_type=jnp.float32)
           + bm2_ref[...])
    out = out * (g_ref[...] * _BN_SCALE) + b_ref[...] + h
    o_ref[...] = jnp.maximum(out, 0.0)


def _layer_tc(h, parts, eps_l, wm1, bm1, wm2, bm2, gamma_l, beta_l):
    return pl.pallas_call(
        _layer_body,
        grid=(N // NB,),
        in_specs=[
            pl.BlockSpec((NB, H), lambda i: (i, 0)),
            pl.BlockSpec((NB, H), lambda i: (i, 0)),
            pl.BlockSpec(memory_space=pltpu.SMEM),
            pl.BlockSpec((H, 2 * H), lambda i: (0, 0)),
            pl.BlockSpec((1, 2 * H), lambda i: (0, 0)),
            pl.BlockSpec((2 * H, H), lambda i: (0, 0)),
            pl.BlockSpec((1, H), lambda i: (0, 0)),
            pl.BlockSpec((1, H), lambda i: (0, 0)),
            pl.BlockSpec((1, H), lambda i: (0, 0)),
        ],
        out_specs=pl.BlockSpec((NB, H), lambda i: (i, 0)),
        out_shape=jax.ShapeDtypeStruct((N, H), jnp.float32),
    )(h, parts, eps_l, wm1, bm1, wm2, bm2, gamma_l, beta_l)


def _pool_body(h_ref, wo1_ref, bo1_ref, wo2_ref, bo2_ref, logits_ref,
               pooled_ref, acc_ref):
    i = pl.program_id(0)

    @pl.when(i == 0)
    def _():
        acc_ref[...] = jnp.zeros_like(acc_ref)

    acc_ref[...] += jnp.sum(h_ref[...], axis=0, keepdims=True)

    @pl.when(i == pl.num_programs(0) - 1)
    def _():
        pooled = acc_ref[...] * (1.0 / N)
        pooled_ref[...] = pooled
        t = jnp.maximum(
            jnp.dot(pooled, wo1_ref[...], preferred_element_type=jnp.float32)
            + bo1_ref[...], 0.0)
        logits_ref[...] = (
            jnp.dot(t, wo2_ref[...], preferred_element_type=jnp.float32)
            + bo2_ref[...])


def _pool_tc(h, wo1, bo1, wo2, bo2):
    h2 = wo1.shape[1]
    nout = wo2.shape[1]
    return pl.pallas_call(
        _pool_body,
        grid=(N // NB,),
        in_specs=[
            pl.BlockSpec((NB, H), lambda i: (i, 0)),
            pl.BlockSpec((H, h2), lambda i: (0, 0)),
            pl.BlockSpec((1, h2), lambda i: (0, 0)),
            pl.BlockSpec((h2, nout), lambda i: (0, 0)),
            pl.BlockSpec((1, nout), lambda i: (0, 0)),
        ],
        out_specs=[
            pl.BlockSpec((1, nout), lambda i: (0, 0)),
            pl.BlockSpec((1, H), lambda i: (0, 0)),
        ],
        out_shape=[
            jax.ShapeDtypeStruct((1, nout), jnp.float32),
            jax.ShapeDtypeStruct((1, H), jnp.float32),
        ],
        scratch_shapes=[pltpu.VMEM((1, H), jnp.float32)],
    )(h, wo1, bo1, wo2, bo2)


def kernel(x, edge_index, edge_attr, W_ne, b_ne, W_ee, b_ee, eps, We, be,
           Wm1, bm1, Wm2, bm2, gamma, beta, Wo1, bo1, Wo2, bo2):
    ei = edge_index.astype(jnp.int32)
    src_r = ei[0].reshape(NW, NGROUPS, GCH, CHUNK)
    dst_r = ei[1].reshape(NW, NGROUPS, GCH, CHUNK)

    h = _node_enc(x, W_ne, b_ne.reshape(1, H))
    e_all = _edge_e(edge_attr, W_ee, We, b_ee.reshape(1, H), be)

    sc_msgpass = _get_sc_msgpass()
    for l in range(NLAYERS):
        parts = sc_msgpass(h, src_r, dst_r, e_all[l]).reshape(N, H)
        h = _layer_tc(h, parts, eps[l].reshape(1, 1), Wm1[l],
                      bm1[l].reshape(1, 2 * H), Wm2[l], bm2[l].reshape(1, H),
                      gamma[l].reshape(1, H), beta[l].reshape(1, H))

    return _pool_tc(h, Wo1, bo1.reshape(1, H // 2), Wo2,
                    bo2.reshape(1, Wo2.shape[1]))


# speedup vs baseline: 3.2010x; 2.3234x over previous
import functools
import math

import jax
import jax.numpy as jnp
from jax import lax
from jax.experimental import pallas as pl
from jax.experimental.pallas import tpu as pltpu
from jax.experimental.pallas import tpu_sc as plsc

N = 10000
E = 320000
H = 128
NLAYERS = 3
NW = 16
EPW = E // NW
CHUNK = 40
NCHUNKS = EPW // CHUNK
GCH = 100
NGROUPS = NCHUNKS // GCH
QUADS = GCH // 4
RPT = N // 16
NB = 1000
EB = 4000

_BN_SCALE = 1.0 / math.sqrt(1.0 + 1e-5)


def _node_enc_body(x_ref, w_ref, b_ref, o_ref):
    o_ref[...] = jnp.maximum(
        jnp.dot(x_ref[...], w_ref[...], preferred_element_type=jnp.float32)
        + b_ref[...], 0.0)


def _node_enc(x, w, b):
    return pl.pallas_call(
        _node_enc_body,
        grid=(N // NB,),
        in_specs=[
            pl.BlockSpec((NB, H), lambda i: (i, 0)),
            pl.BlockSpec((H, H), lambda i: (0, 0)),
            pl.BlockSpec((1, H), lambda i: (0, 0)),
        ],
        out_specs=pl.BlockSpec((NB, H), lambda i: (i, 0)),
        out_shape=jax.ShapeDtypeStruct((N, H), jnp.float32),
    )(x, w, b)


def _edge_e_body(ea_ref, wee_ref, we_ref, bee_ref, be_ref, e0_ref, e1_ref,
                 e2_ref):
    ea = ea_ref[...]
    outs = (e0_ref, e1_ref, e2_ref)
    for l in range(NLAYERS):
        wl = we_ref[l]
        wp = jnp.dot(wee_ref[...], wl, preferred_element_type=jnp.float32)
        bp = (jnp.dot(bee_ref[...], wl, preferred_element_type=jnp.float32)
              + be_ref[l][None, :])
        outs[l][...] = jnp.dot(ea, wp, preferred_element_type=jnp.float32) + bp


def _edge_e(edge_attr, w_ee, we, b_ee, be):
    d_edge = edge_attr.shape[1]
    espec = pl.BlockSpec((EB, H), lambda i: (i, 0))
    return pl.pallas_call(
        _edge_e_body,
        grid=(E // EB,),
        in_specs=[
            pl.BlockSpec((EB, d_edge), lambda i: (i, 0)),
            pl.BlockSpec((d_edge, H), lambda i: (0, 0)),
            pl.BlockSpec((NLAYERS, H, H), lambda i: (0, 0, 0)),
            pl.BlockSpec((1, H), lambda i: (0, 0)),
            pl.BlockSpec((NLAYERS, H), lambda i: (0, 0)),
        ],
        out_specs=[espec, espec, espec],
        out_shape=[jax.ShapeDtypeStruct((E, H), jnp.float32)] * NLAYERS,
    )(edge_attr, w_ee, we, b_ee, be)


@functools.cache
def _get_sc_msgpass():
    mesh = plsc.VectorSubcoreMesh(core_axis_name="c", subcore_axis_name="s",
                                  num_cores=1)
    return functools.partial(
        pl.kernel,
        out_type=jax.ShapeDtypeStruct((16, RPT, H), jnp.float32),
        mesh=mesh,
        scratch_types=[
            pltpu.VMEM((GCH, CHUNK), jnp.int32),
            pltpu.VMEM((GCH, CHUNK), jnp.int32),
            pltpu.VMEM((CHUNK, H), jnp.float32),
            pltpu.VMEM((CHUNK, H), jnp.float32),
            pltpu.VMEM((CHUNK, H), jnp.float32),
            pltpu.VMEM((CHUNK, H), jnp.float32),
            pltpu.VMEM_SHARED((N, H), jnp.float32),
        ] + [pltpu.SemaphoreType.DMA] * 12,
    )(_sc_msgpass_body)


def _sc_msgpass_body(h_hbm, src_hbm, dst_hbm, e_hbm, out_hbm, srcs_v, dsts_v,
                     buf0, buf1, buf2, buf3, aggr_sh,
                     es0, es1, es2, es3, gs0, gs1, gs2, gs3,
                     ss0, ss1, ss2, ss3):
    sid = lax.axis_index("s")
    wid = sid
    bufs = (buf0, buf1, buf2, buf3)
    esems = (es0, es1, es2, es3)
    gsems = (gs0, gs1, gs2, gs3)
    ssems = (ss0, ss1, ss2, ss3)

    def _zrow(r, carry):
        for j in range(H // 16):
            buf0[r, pl.ds(j * 16, 16)] = jnp.zeros((16,), jnp.float32)
        return carry
    lax.fori_loop(0, CHUNK, _zrow, 0)

    def _zcopy(k, carry):
        pltpu.sync_copy(buf0,
                        aggr_sh.at[pl.ds(sid * RPT + k * CHUNK, CHUNK), :])
        return carry
    lax.fori_loop(0, RPT // CHUNK, _zcopy, 0)
    _ztail = RPT - (RPT // CHUNK) * CHUNK
    pltpu.sync_copy(
        buf0.at[pl.ds(0, _ztail), :],
        aggr_sh.at[pl.ds(sid * RPT + (RPT // CHUNK) * CHUNK, _ztail), :])
    plsc.subcore_barrier()

    ebase = wid * EPW

    def _relu(b):
        def _rrow(r, rc):
            for k in range(H // 16):
                sl = pl.ds(k * 16, 16)
                bufs[b][r, sl] = jnp.maximum(bufs[b][r, sl], 0.0)
            return rc
        lax.fori_loop(0, CHUNK, _rrow, 0)

    def _group(gg, gcarry):
        pltpu.sync_copy(src_hbm.at[wid, gg], srcs_v)
        pltpu.sync_copy(dst_hbm.at[wid, gg], dsts_v)
        gbase = ebase + gg * (GCH * CHUNK)

        def e_start(b, l):
            pltpu.async_copy(e_hbm.at[pl.ds(gbase + l * CHUNK, CHUNK), :],
                             bufs[b], esems[b])

        def e_wait(b, l):
            pltpu.make_async_copy(
                e_hbm.at[pl.ds(gbase + l * CHUNK, CHUNK), :],
                bufs[b], esems[b]).wait()

        def g_start(b, l):
            pltpu.async_copy(h_hbm.at[srcs_v.at[l]], bufs[b], gsems[b],
                             add=True)

        def g_wait(b, l):
            pltpu.make_async_copy(h_hbm.at[srcs_v.at[l]], bufs[b],
                                  gsems[b]).wait()

        def s_start(b, l):
            pltpu.async_copy(bufs[b], aggr_sh.at[dsts_v.at[l]], ssems[b],
                             add=True)

        def s_wait(b, l):
            pltpu.make_async_copy(bufs[b], aggr_sh.at[dsts_v.at[l]],
                                  ssems[b]).wait()

        e_start(0, 0)
        e_start(1, 1)
        e_wait(0, 0)
        g_start(0, 0)

        def _quad(g, qcarry):
            l0 = g * 4
            @pl.when(g >= 1)
            def _():
                s_wait(2, l0 - 2)
            e_start(2, l0 + 2)
            e_wait(1, l0 + 1)
            g_start(1, l0 + 1)
            g_wait(0, l0)
            _relu(0)
            s_start(0, l0)
            @pl.when(g >= 1)
            def _():
                s_wait(3, l0 - 1)
            e_start(3, l0 + 3)
            e_wait(2, l0 + 2)
            g_start(2, l0 + 2)
            g_wait(1, l0 + 1)
            _relu(1)
            s_start(1, l0 + 1)
            s_wait(0, l0)
            @pl.when(g < QUADS - 1)
            def _():
                e_start(0, l0 + 4)
            e_wait(3, l0 + 3)
            g_start(3, l0 + 3)
            g_wait(2, l0 + 2)
            _relu(2)
            s_start(2, l0 + 2)
            s_wait(1, l0 + 1)
            @pl.when(g < QUADS - 1)
            def _():
                e_start(1, l0 + 5)
                e_wait(0, l0 + 4)
                g_start(0, l0 + 4)
            g_wait(3, l0 + 3)
            _relu(3)
            s_start(3, l0 + 3)
            return qcarry

        lax.fori_loop(0, QUADS, _quad, 0)
        s_wait(2, GCH - 2)
        s_wait(3, GCH - 1)
        return gcarry

    lax.fori_loop(0, NGROUPS, _group, 0)
    plsc.subcore_barrier()

    pltpu.sync_copy(aggr_sh.at[pl.ds(sid * RPT, RPT), :], out_hbm.at[sid])


def _layer_body(h_ref, p_ref, eps_ref, wm1_ref, bm1_ref, wm2_ref, bm2_ref,
                g_ref, b_ref, o_ref):
    h = h_ref[...]
    out = (1.0 + eps_ref[0, 0]) * h + p_ref[...]
    t = jnp.maximum(
        jnp.dot(out, wm1_ref[...], preferred_element_type=jnp.float32)
        + bm1_ref[...], 0.0)
    out = (jnp.dot(t, wm2_ref[...], preferred_element_type=jnp.float32)
           + bm2_ref[...])
    out = out * (g_ref[...] * _BN_SCALE) + b_ref[...] + h
    o_ref[...] = jnp.maximum(out, 0.0)


def _layer_tc(h, parts, eps_l, wm1, bm1, wm2, bm2, gamma_l, beta_l):
    return pl.pallas_call(
        _layer_body,
        grid=(N // NB,),
        in_specs=[
            pl.BlockSpec((NB, H), lambda i: (i, 0)),
            pl.BlockSpec((NB, H), lambda i: (i, 0)),
            pl.BlockSpec(memory_space=pltpu.SMEM),
            pl.BlockSpec((H, 2 * H), lambda i: (0, 0)),
            pl.BlockSpec((1, 2 * H), lambda i: (0, 0)),
            pl.BlockSpec((2 * H, H), lambda i: (0, 0)),
            pl.BlockSpec((1, H), lambda i: (0, 0)),
            pl.BlockSpec((1, H), lambda i: (0, 0)),
            pl.BlockSpec((1, H), lambda i: (0, 0)),
        ],
        out_specs=pl.BlockSpec((NB, H), lambda i: (i, 0)),
        out_shape=jax.ShapeDtypeStruct((N, H), jnp.float32),
    )(h, parts, eps_l, wm1, bm1, wm2, bm2, gamma_l, beta_l)


def _pool_body(h_ref, wo1_ref, bo1_ref, wo2_ref, bo2_ref, logits_ref,
               pooled_ref, acc_ref):
    i = pl.program_id(0)

    @pl.when(i == 0)
    def _():
        acc_ref[...] = jnp.zeros_like(acc_ref)

    acc_ref[...] += jnp.sum(h_ref[...], axis=0, keepdims=True)

    @pl.when(i == pl.num_programs(0) - 1)
    def _():
        pooled = acc_ref[...] * (1.0 / N)
        pooled_ref[...] = pooled
        t = jnp.maximum(
            jnp.dot(pooled, wo1_ref[...], preferred_element_type=jnp.float32)
            + bo1_ref[...], 0.0)
        logits_ref[...] = (
            jnp.dot(t, wo2_ref[...], preferred_element_type=jnp.float32)
            + bo2_ref[...])


def _pool_tc(h, wo1, bo1, wo2, bo2):
    h2 = wo1.shape[1]
    nout = wo2.shape[1]
    return pl.pallas_call(
        _pool_body,
        grid=(N // NB,),
        in_specs=[
            pl.BlockSpec((NB, H), lambda i: (i, 0)),
            pl.BlockSpec((H, h2), lambda i: (0, 0)),
            pl.BlockSpec((1, h2), lambda i: (0, 0)),
            pl.BlockSpec((h2, nout), lambda i: (0, 0)),
            pl.BlockSpec((1, nout), lambda i: (0, 0)),
        ],
        out_specs=[
            pl.BlockSpec((1, nout), lambda i: (0, 0)),
            pl.BlockSpec((1, H), lambda i: (0, 0)),
        ],
        out_shape=[
            jax.ShapeDtypeStruct((1, nout), jnp.float32),
            jax.ShapeDtypeStruct((1, H), jnp.float32),
        ],
        scratch_shapes=[pltpu.VMEM((1, H), jnp.float32)],
    )(h, wo1, bo1, wo2, bo2)


def kernel(x, edge_index, edge_attr, W_ne, b_ne, W_ee, b_ee, eps, We, be,
           Wm1, bm1, Wm2, bm2, gamma, beta, Wo1, bo1, Wo2, bo2):
    ei = edge_index.astype(jnp.int32)
    src_r = ei[0].reshape(NW, NGROUPS, GCH, CHUNK)
    dst_r = ei[1].reshape(NW, NGROUPS, GCH, CHUNK)

    h = _node_enc(x, W_ne, b_ne.reshape(1, H))
    e_all = _edge_e(edge_attr, W_ee, We, b_ee.reshape(1, H), be)

    sc_msgpass = _get_sc_msgpass()
    for l in range(NLAYERS):
        parts = sc_msgpass(h, src_r, dst_r, e_all[l]).reshape(N, H)
        h = _layer_tc(h, parts, eps[l].reshape(1, 1), Wm1[l],
                      bm1[l].reshape(1, 2 * H), Wm2[l], bm2[l].reshape(1, H),
                      gamma[l].reshape(1, H), beta[l].reshape(1, H))

    return _pool_tc(h, Wo1, bo1.reshape(1, H // 2), Wo2,
                    bo2.reshape(1, Wo2.shape[1]))
